# Initial kernel scaffold; baseline (speedup 1.0000x reference)
import jax
import jax.numpy as jnp
from jax.experimental import pallas as pl


def kernel(outputs, target, bins):
    raise NotImplementedError("write your pallas kernel here")


# speedup vs baseline: 33.1397x; 33.1397x over previous
import functools

import jax
import jax.numpy as jnp
from jax.experimental import pallas as pl


def _nll_block_kernel(out_ref, tgt_ref, bins_ref, nll_ref):
    x = out_ref[...]
    t = tgt_ref[...]
    b = bins_ref[...]

    m = jnp.max(x, axis=1, keepdims=True)
    lse = m + jnp.log(jnp.sum(jnp.exp(x - m), axis=1, keepdims=True))

    cnt = jnp.sum((b < t).astype(jnp.int32), axis=1, keepdims=True)
    idx = jnp.clip(cnt - 1, 0, 63)

    logw = jnp.log(b[:, 1:65] - b[:, 0:64])
    iota = jax.lax.broadcasted_iota(jnp.int32, x.shape, 1)
    onehot = iota == idx
    picked = jnp.sum(jnp.where(onehot, x - logw, 0.0), axis=1, keepdims=True)

    nll_ref[...] = lse - picked


@jax.jit
def kernel(outputs, target, bins):
    n, k = outputs.shape
    rows_per_block = 2048
    grid = n // rows_per_block

    bins_pad = jnp.concatenate(
        [bins, jnp.full((128 - bins.shape[0],), jnp.inf, dtype=bins.dtype)]
    ).reshape(1, 128)
    target2 = target.reshape(n, 1)

    nll = pl.pallas_call(
        _nll_block_kernel,
        grid=(grid,),
        in_specs=[
            pl.BlockSpec((rows_per_block, k), lambda i: (i, 0)),
            pl.BlockSpec((rows_per_block, 1), lambda i: (i, 0)),
            pl.BlockSpec((1, 128), lambda i: (0, 0)),
        ],
        out_specs=pl.BlockSpec((rows_per_block, 1), lambda i: (i, 0)),
        out_shape=jax.ShapeDtypeStruct((n, 1), outputs.dtype),
    )(outputs, target2, bins_pad)
    return nll.reshape(n)


# speedup vs baseline: 35.0260x; 1.0569x over previous
import jax
import jax.numpy as jnp
from jax.experimental import pallas as pl


def _nll_block_kernel(x_ref, tgt_ref, lo_ref, hi_ref, lw_ref, nll_ref):
    x = x_ref[...]
    t = tgt_ref[...]
    lo = lo_ref[...]
    hi = hi_ref[...]
    lw = lw_ref[...]

    lane = jax.lax.broadcasted_iota(jnp.int32, x.shape, 1)
    in_a = lane < 64
    t0 = t[:, 0:1]
    t1 = t[:, 1:2]
    tvec = jnp.where(in_a, t0, t1)

    neg = jnp.float32(-jnp.inf)
    ma = jnp.max(jnp.where(in_a, x, neg), axis=1, keepdims=True)
    mb = jnp.max(jnp.where(in_a, neg, x), axis=1, keepdims=True)
    mvec = jnp.where(in_a, ma, mb)
    e = jnp.exp(x - mvec)
    s_all = jnp.sum(e, axis=1, keepdims=True)
    s_a = jnp.sum(jnp.where(in_a, e, 0.0), axis=1, keepdims=True)
    s_b = s_all - s_a

    onehot = (lo < tvec) & ~(hi < tvec)
    contrib = jnp.where(onehot, x - lw, 0.0)
    p_all = jnp.sum(contrib, axis=1, keepdims=True)
    p_a = jnp.sum(jnp.where(in_a, contrib, 0.0), axis=1, keepdims=True)
    p_b = p_all - p_a

    nll_a = ma + jnp.log(s_a) - p_a
    nll_b = mb + jnp.log(s_b) - p_b
    nll_ref[...] = jnp.concatenate([nll_a, nll_b], axis=1)


@jax.jit
def kernel(outputs, target, bins):
    n, k = outputs.shape
    n2 = n // 2
    rows_per_block = 1024
    grid = n2 // rows_per_block

    inf = jnp.inf
    lo = bins[0:64].at[0].set(-inf)
    hi = bins[1:65].at[63].set(inf)
    lw = jnp.log(bins[1:65] - bins[0:64])
    lo2 = jnp.concatenate([lo, lo]).reshape(1, 128)
    hi2 = jnp.concatenate([hi, hi]).reshape(1, 128)
    lw2 = jnp.concatenate([lw, lw]).reshape(1, 128)

    x2 = outputs.reshape(n2, 128)
    t2 = target.reshape(n2, 2)

    nll = pl.pallas_call(
        _nll_block_kernel,
        grid=(grid,),
        in_specs=[
            pl.BlockSpec((rows_per_block, 128), lambda i: (i, 0)),
            pl.BlockSpec((rows_per_block, 2), lambda i: (i, 0)),
            pl.BlockSpec((1, 128), lambda i: (0, 0)),
            pl.BlockSpec((1, 128), lambda i: (0, 0)),
            pl.BlockSpec((1, 128), lambda i: (0, 0)),
        ],
        out_specs=pl.BlockSpec((rows_per_block, 2), lambda i: (i, 0)),
        out_shape=jax.ShapeDtypeStruct((n2, 2), outputs.dtype),
    )(x2, t2, lo2, hi2, lw2)
    return nll.reshape(n)


# speedup vs baseline: 84.9993x; 2.4267x over previous
import jax
import jax.numpy as jnp
from jax.experimental import pallas as pl

_BS = 16


def _nll_block_kernel(x_ref, tgt_ref, lo_ref, hi_ref, lw_ref, nll_ref):
    x = x_ref[...]
    t2 = tgt_ref[...]
    lo = lo_ref[...].reshape(1, 64, 128)
    hi = hi_ref[...].reshape(1, 64, 128)
    lw = lw_ref[...].reshape(1, 64, 128)

    xt = jax.lax.transpose(x, (0, 2, 1))
    t3 = t2.reshape(_BS, 1, 128)

    m = jnp.max(xt, axis=1, keepdims=True)
    e = jnp.exp(xt - m)
    s = jnp.sum(e, axis=1, keepdims=True)

    onehot = (lo < t3) & ~(hi < t3)
    picked = jnp.sum(jnp.where(onehot, xt - lw, 0.0), axis=1, keepdims=True)

    nll = m + jnp.log(s) - picked
    nll_ref[...] = nll.reshape(_BS, 128)


@jax.jit
def kernel(outputs, target, bins):
    n, k = outputs.shape
    rows = n // 128
    grid = rows // _BS

    inf = jnp.inf
    lo = bins[0:64].at[0].set(-inf)
    hi = bins[1:65].at[63].set(inf)
    lw = jnp.log(bins[1:65] - bins[0:64])
    ones = jnp.ones((1, 128), dtype=bins.dtype)
    lo2 = lo.reshape(64, 1) * ones
    hi2 = hi.reshape(64, 1) * ones
    lw2 = lw.reshape(64, 1) * ones

    x3 = outputs.reshape(rows, 128, k)
    t2 = target.reshape(rows, 128)

    nll = pl.pallas_call(
        _nll_block_kernel,
        grid=(grid,),
        in_specs=[
            pl.BlockSpec((_BS, 128, k), lambda i: (i, 0, 0)),
            pl.BlockSpec((_BS, 128), lambda i: (i, 0)),
            pl.BlockSpec((64, 128), lambda i: (0, 0)),
            pl.BlockSpec((64, 128), lambda i: (0, 0)),
            pl.BlockSpec((64, 128), lambda i: (0, 0)),
        ],
        out_specs=pl.BlockSpec((_BS, 128), lambda i: (i, 0)),
        out_shape=jax.ShapeDtypeStruct((rows, 128), outputs.dtype),
    )(x3, t2, lo2, hi2, lw2)
    return nll.reshape(n)


# speedup vs baseline: 118.7862x; 1.3975x over previous
import jax
import jax.numpy as jnp
from jax.experimental import pallas as pl

_BS = 64


def _nll_block_kernel(x_ref, tgt_ref, lo_ref, hi_ref, lw_ref, nll_ref):
    x = x_ref[...]
    t2 = tgt_ref[...]
    lo = lo_ref[...].reshape(1, 64, 128)
    hi = hi_ref[...].reshape(1, 64, 128)
    lw = lw_ref[...].reshape(1, 64, 128)

    xt = jax.lax.transpose(x, (0, 2, 1))
    t3 = t2.reshape(_BS, 1, 128)

    m = jnp.max(xt, axis=1, keepdims=True)
    e = jnp.exp(xt - m)
    s = jnp.sum(e, axis=1, keepdims=True)

    onehot = (lo < t3) & ~(hi < t3)
    picked = jnp.sum(jnp.where(onehot, xt - lw, 0.0), axis=1, keepdims=True)

    nll = m + jnp.log(s) - picked
    nll_ref[...] = nll.reshape(_BS, 128)


@jax.jit
def kernel(outputs, target, bins):
    n, k = outputs.shape
    rows = n // 128
    grid = rows // _BS

    inf = jnp.inf
    lo = bins[0:64].at[0].set(-inf)
    hi = bins[1:65].at[63].set(inf)
    lw = jnp.log(bins[1:65] - bins[0:64])
    ones = jnp.ones((1, 128), dtype=bins.dtype)
    lo2 = lo.reshape(64, 1) * ones
    hi2 = hi.reshape(64, 1) * ones
    lw2 = lw.reshape(64, 1) * ones

    x3 = outputs.reshape(rows, 128, k)
    t2 = target.reshape(rows, 128)

    nll = pl.pallas_call(
        _nll_block_kernel,
        grid=(grid,),
        in_specs=[
            pl.BlockSpec((_BS, 128, k), lambda i: (i, 0, 0)),
            pl.BlockSpec((_BS, 128), lambda i: (i, 0)),
            pl.BlockSpec((64, 128), lambda i: (0, 0)),
            pl.BlockSpec((64, 128), lambda i: (0, 0)),
            pl.BlockSpec((64, 128), lambda i: (0, 0)),
        ],
        out_specs=pl.BlockSpec((_BS, 128), lambda i: (i, 0)),
        out_shape=jax.ShapeDtypeStruct((rows, 128), outputs.dtype),
    )(x3, t2, lo2, hi2, lw2)
    return nll.reshape(n)


# speedup vs baseline: 129.1223x; 1.0870x over previous
import jax
import jax.numpy as jnp
from jax.experimental import pallas as pl

_BS = 256


def _nll_block_kernel(x_ref, tgt_ref, lo_ref, hi_ref, lw_ref, nll_ref):
    x = x_ref[...]
    t2 = tgt_ref[...]
    lo = lo_ref[...].reshape(1, 64, 128)
    hi = hi_ref[...].reshape(1, 64, 128)
    lw = lw_ref[...].reshape(1, 64, 128)

    xt = jax.lax.transpose(x, (0, 2, 1))
    t3 = t2.reshape(_BS, 1, 128)

    m = jnp.max(xt, axis=1, keepdims=True)
    e = jnp.exp(xt - m)
    s = jnp.sum(e, axis=1, keepdims=True)

    onehot = (lo < t3) & ~(hi < t3)
    picked = jnp.sum(jnp.where(onehot, xt - lw, 0.0), axis=1, keepdims=True)

    nll = m + jnp.log(s) - picked
    nll_ref[...] = nll.reshape(_BS, 128)


@jax.jit
def kernel(outputs, target, bins):
    n, k = outputs.shape
    rows = n // 128
    grid = rows // _BS

    inf = jnp.inf
    lo = bins[0:64].at[0].set(-inf)
    hi = bins[1:65].at[63].set(inf)
    lw = jnp.log(bins[1:65] - bins[0:64])
    ones = jnp.ones((1, 128), dtype=bins.dtype)
    lo2 = lo.reshape(64, 1) * ones
    hi2 = hi.reshape(64, 1) * ones
    lw2 = lw.reshape(64, 1) * ones

    x3 = outputs.reshape(rows, 128, k)
    t2 = target.reshape(rows, 128)

    nll = pl.pallas_call(
        _nll_block_kernel,
        grid=(grid,),
        in_specs=[
            pl.BlockSpec((_BS, 128, k), lambda i: (i, 0, 0)),
            pl.BlockSpec((_BS, 128), lambda i: (i, 0)),
            pl.BlockSpec((64, 128), lambda i: (0, 0)),
            pl.BlockSpec((64, 128), lambda i: (0, 0)),
            pl.BlockSpec((64, 128), lambda i: (0, 0)),
        ],
        out_specs=pl.BlockSpec((_BS, 128), lambda i: (i, 0)),
        out_shape=jax.ShapeDtypeStruct((rows, 128), outputs.dtype),
    )(x3, t2, lo2, hi2, lw2)
    return nll.reshape(n)
